# baseline (speedup 1.0000x reference)
import functools

import jax
import jax.numpy as jnp
from jax import lax
from jax.experimental import pallas as pl
from jax.experimental.pallas import tpu as pltpu
from jax.experimental.pallas import tpu_sc as plsc

EPS = 0.1
N_ITERS = 50
K = 8.0

B, N, D = 4, 8192, 768
KSEL = 512
N_CHUNK = 1024


def _matvec_body(x_ref, rt_ref, s_ref):
    x = x_ref[:, :, :].reshape(B * N_CHUNK, D)
    rt = rt_ref[0]
    s = jax.lax.dot_general(
        x, rt[:, None],
        dimension_numbers=(((1,), (0,)), ((), ())),
        preferred_element_type=jnp.float32,
        precision=jax.lax.Precision.DEFAULT,
    )
    s_ref[:, :] = s.reshape(B, N_CHUNK)


def _iters_body(s_ref, a_ref):
    s = s_ref[:, :]
    constant = EPS * jnp.log(K)
    inv_eps = jnp.float32(1.0 / EPS)

    def one_iter(_, a):
        u = jnp.minimum(s + a, 0.0) * inv_eps
        ssum = jnp.sum(jnp.exp(u), axis=-1, keepdims=True)
        return a + (constant - EPS * jnp.log(ssum))

    a = jax.lax.fori_loop(
        0, N_ITERS, one_iter, jnp.zeros((B, 1), jnp.float32))
    a_ref[:, :] = a


def _cumsum_lanes(x):
    n = x.shape[1]
    sh = 1
    while sh < n:
        shifted = jnp.concatenate(
            [jnp.zeros((x.shape[0], sh), x.dtype), x[:, :n - sh]], axis=1)
        x = x + shifted
        sh *= 2
    return x


def _topk_body(sc_ref, idx_ref, work_ref, ceq_ref):
    sc = sc_ref[:, :]
    sb = jax.lax.bitcast_convert_type(sc, jnp.int32)
    iota_n = jax.lax.broadcasted_iota(jnp.int32, (B, N), 1)

    cnt_pos = jnp.sum(jnp.where(sb > 0, 1, 0), axis=1, keepdims=True)

    def bisect_all(_):
        def bisect_step(_, carry):
            lo, hi = carry
            mid = (lo + hi) // 2
            cnt = jnp.sum(jnp.where(sb >= mid, 1, 0), axis=1, keepdims=True)
            ok = cnt >= KSEL
            return (jnp.where(ok, mid, lo), jnp.where(ok, hi, mid))

        lo0 = jnp.zeros((B, 1), jnp.int32)
        hi0 = jnp.full((B, 1), 0x3F800001, jnp.int32)
        lo, _ = jax.lax.fori_loop(0, 31, bisect_step, (lo0, hi0))
        return lo

    t = jax.lax.cond(
        jnp.all(cnt_pos < KSEL),
        lambda _: jnp.zeros((B, 1), jnp.int32),
        bisect_all,
        operand=0,
    )

    mask_gt = sb > t
    g = jnp.sum(jnp.where(mask_gt, 1, 0), axis=1, keepdims=True)

    work_ref[:, :] = jnp.where(mask_gt, sc, -1.0)
    max_g = jnp.max(g)
    slot_iota = jax.lax.broadcasted_iota(jnp.int32, (B, KSEL), 1)

    def extract_step(p, gslot):
        w = work_ref[:, :]
        m = jnp.max(w, axis=1, keepdims=True)
        amin = jnp.min(jnp.where(w == m, iota_n, N), axis=1, keepdims=True)
        work_ref[:, :] = jnp.where(iota_n == amin, -1.0, w)
        return jnp.where(slot_iota == p, amin, gslot)

    gslot = jax.lax.fori_loop(
        0, max_g, extract_step, jnp.zeros((B, KSEL), jnp.int32))

    ceq_ref[:, :] = _cumsum_lanes(jnp.where(sb == t, 1, 0))
    lim = slot_iota - g
    CHN = 512

    def nchunk_step(j, acc):
        cc = ceq_ref[:, pl.ds(j * CHN, CHN)]
        part = jnp.sum(
            jnp.where(cc[:, None, :] <= lim[:, :, None], 1, 0), axis=2)
        return acc + part

    pmax = jnp.max(jnp.sum(jnp.where(ceq_ref[:, :] <= KSEL, 1, 0), axis=1))
    nchunks = (pmax + CHN - 1) // CHN
    eqpos = jax.lax.fori_loop(
        0, nchunks, nchunk_step, jnp.zeros((B, KSEL), jnp.int32))

    idx_ref[:, :] = jnp.where(slot_iota < g, gslot, eqpos)


_SC_L = 16
_SC_NV = N // _SC_L


def _sc_topk_body(sc_hbm, out_hbm, row_v, candv_v, candi_v, out_v):
    wid = lax.axis_index("s") * 2 + lax.axis_index("c")

    @pl.when(wid < B)
    def _():
        pltpu.sync_copy(sc_hbm.at[wid], row_v)
        iota16 = lax.broadcasted_iota(jnp.int32, (_SC_L,), 0)

        def vload(i):
            return row_v[pl.ds(i * _SC_L, _SC_L)]

        def cursor_max(vals_ref, nv, lb, li):
            def scan_body(j, acc):
                ab, ai = acc
                v = vals_ref[pl.ds(j * _SC_L, _SC_L)]
                iv = iota16 + j * _SC_L
                elig = (v < lb) | ((v == lb) & (iv > li))
                v = jnp.where(elig, v, -1.0)
                iv = jnp.where(elig, iv, N)
                better = (v > ab) | ((v == ab) & (iv < ai))
                return (jnp.where(better, v, ab), jnp.where(better, iv, ai))

            ab, ai = lax.fori_loop(
                0, nv, scan_body,
                (jnp.full((_SC_L,), -1.0, jnp.float32),
                 jnp.full((_SC_L,), N, jnp.int32)))
            mb = jnp.max(ab)
            mi = jnp.min(jnp.where(ab == mb, ai, N))
            return mb, mi

        def store_slot(p, val):
            plsc.store_scatter(out_v, [jnp.full((_SC_L,), p, jnp.int32)],
                               jnp.full((_SC_L,), val, jnp.int32),
                               mask=iota16 == 0)

        def cnt_body(i, acc):
            return acc + jnp.where(vload(i) > 0.0, 1, 0)

        cnt_pos = jnp.sum(
            lax.fori_loop(0, _SC_NV, cnt_body, jnp.zeros((_SC_L,), jnp.int32)))

        @pl.when(cnt_pos < KSEL)
        def fast_path():
            def comp_body(i, base):
                v = vload(i)
                m = v > 0.0
                pc = plsc.cumsum(jnp.where(m, 1, 0))
                dest = base + pc - 1
                plsc.store_scatter(candv_v, [dest], v, mask=m)
                plsc.store_scatter(candi_v, [dest], iota16 + i * _SC_L,
                                   mask=m)
                return base + jnp.sum(jnp.where(m, 1, 0))

            g = lax.fori_loop(0, _SC_NV, comp_body, 0)

            def pad_body(i, _):
                dest = iota16 + i * _SC_L
                m = dest >= g
                plsc.store_scatter(candv_v, [dest],
                                   jnp.full((_SC_L,), -1.0, jnp.float32),
                                   mask=m)
                return 0

            lax.fori_loop(0, KSEL // _SC_L, pad_body, 0)

            def cext_body(p, carry):
                lb, li = carry

                def scan_body(j, acc):
                    ab, ai = acc
                    v = candv_v[pl.ds(j * _SC_L, _SC_L)]
                    iv = candi_v[pl.ds(j * _SC_L, _SC_L)]
                    elig = (v < lb) | ((v == lb) & (iv > li))
                    v = jnp.where(elig, v, -1.0)
                    iv = jnp.where(elig, iv, N)
                    better = (v > ab) | ((v == ab) & (iv < ai))
                    return (jnp.where(better, v, ab),
                            jnp.where(better, iv, ai))

                ab, ai = lax.fori_loop(
                    0, KSEL // _SC_L, scan_body,
                    (jnp.full((_SC_L,), -1.0, jnp.float32),
                     jnp.full((_SC_L,), N, jnp.int32)))
                mb = jnp.max(ab)
                mi = jnp.min(jnp.where(ab == mb, ai, N))
                store_slot(p, mi)
                return (mb, mi)

            lax.fori_loop(0, g, cext_body, (jnp.float32(2.0), -1))

            def eq_cond(carry):
                i, cnt = carry
                return (i < _SC_NV) & (cnt < KSEL - g)

            def eq_body(carry):
                i, cnt = carry
                m = vload(i) == 0.0
                pc = plsc.cumsum(jnp.where(m, 1, 0))
                dest = g + cnt + pc - 1
                plsc.store_scatter(out_v, [dest], iota16 + i * _SC_L,
                                   mask=m & (dest < KSEL))
                return (i + 1, cnt + jnp.sum(jnp.where(m, 1, 0)))

            lax.while_loop(eq_cond, eq_body, (0, 0))

        @pl.when(cnt_pos >= KSEL)
        def general_path():
            def ext_body(p, carry):
                lb, li = carry
                mb, mi = cursor_max(row_v, _SC_NV, lb, li)
                store_slot(p, mi)
                return (mb, mi)

            lax.fori_loop(0, KSEL, ext_body, (jnp.float32(2.0), -1))

        pltpu.sync_copy(out_v, out_hbm.at[wid])


def _sc_topk(scores):
    return pl.kernel(
        _sc_topk_body,
        out_type=jax.ShapeDtypeStruct((B, KSEL), jnp.int32),
        mesh=plsc.VectorSubcoreMesh(core_axis_name="c", subcore_axis_name="s"),
        compiler_params=pltpu.CompilerParams(needs_layout_passes=False),
        scratch_types=[
            pltpu.VMEM((N,), jnp.float32),
            pltpu.VMEM((KSEL,), jnp.float32),
            pltpu.VMEM((KSEL,), jnp.int32),
            pltpu.VMEM((KSEL,), jnp.int32),
        ],
    )(scores)


def _compute(x, rt):
    s = pl.pallas_call(
        _matvec_body,
        grid=(N // N_CHUNK,),
        in_specs=[
            pl.BlockSpec((B, N_CHUNK, D), lambda j: (0, j, 0)),
            pl.BlockSpec((1, D), lambda j: (0, 0)),
        ],
        out_specs=pl.BlockSpec((B, N_CHUNK), lambda j: (0, j)),
        out_shape=jax.ShapeDtypeStruct((B, N), jnp.float32),
    )(x, rt[None, :])

    a = pl.pallas_call(
        _iters_body,
        out_shape=jax.ShapeDtypeStruct((B, 1), jnp.float32),
    )(s)

    bfin = -jax.nn.relu(s + a)
    scores = jnp.exp((s + a + bfin) / EPS)

    return _sc_topk(scores)


def kernel(x, routing_token, num_tokens):
    n = x.shape[-2]
    num_tokens = jnp.minimum(num_tokens, n)
    idx = _compute(x, routing_token)
    valid = jnp.arange(KSEL) < num_tokens
    sel_scores = jnp.broadcast_to(
        jnp.where(valid, 1.0, 0.0).astype(jnp.float32), (x.shape[0], KSEL))
    sel_idx = jnp.where(valid, idx, 0)
    return (sel_scores, sel_idx)


# speedup vs baseline: 1.0056x; 1.0056x over previous
import functools

import jax
import jax.numpy as jnp
from jax import lax
from jax.experimental import pallas as pl
from jax.experimental.pallas import tpu as pltpu
from jax.experimental.pallas import tpu_sc as plsc

EPS = 0.1
N_ITERS = 50
K = 8.0

B, N, D = 4, 8192, 768
KSEL = 512
N_CHUNK = 1024


def _matvec_body(x_ref, rt_ref, s_ref):
    x = x_ref[:, :, :].reshape(B * N_CHUNK, D)
    rt = rt_ref[0]
    s = jax.lax.dot_general(
        x, rt[:, None],
        dimension_numbers=(((1,), (0,)), ((), ())),
        preferred_element_type=jnp.float32,
        precision=jax.lax.Precision.DEFAULT,
    )
    s_ref[:, :] = s.reshape(B, N_CHUNK)


def _iters_body(s_ref, a_ref):
    s = s_ref[:, :]
    constant = EPS * jnp.log(K)
    inv_eps = jnp.float32(1.0 / EPS)

    def one_iter(_, a):
        u = jnp.minimum(s + a, 0.0) * inv_eps
        ssum = jnp.sum(jnp.exp(u), axis=-1, keepdims=True)
        return a + (constant - EPS * jnp.log(ssum))

    a = jax.lax.fori_loop(
        0, N_ITERS, one_iter, jnp.zeros((B, 1), jnp.float32))
    a_ref[:, :] = a


def _cumsum_lanes(x):
    n = x.shape[1]
    sh = 1
    while sh < n:
        shifted = jnp.concatenate(
            [jnp.zeros((x.shape[0], sh), x.dtype), x[:, :n - sh]], axis=1)
        x = x + shifted
        sh *= 2
    return x


def _topk_body(sc_ref, idx_ref, work_ref, ceq_ref):
    sc = sc_ref[:, :]
    sb = jax.lax.bitcast_convert_type(sc, jnp.int32)
    iota_n = jax.lax.broadcasted_iota(jnp.int32, (B, N), 1)

    cnt_pos = jnp.sum(jnp.where(sb > 0, 1, 0), axis=1, keepdims=True)

    def bisect_all(_):
        def bisect_step(_, carry):
            lo, hi = carry
            mid = (lo + hi) // 2
            cnt = jnp.sum(jnp.where(sb >= mid, 1, 0), axis=1, keepdims=True)
            ok = cnt >= KSEL
            return (jnp.where(ok, mid, lo), jnp.where(ok, hi, mid))

        lo0 = jnp.zeros((B, 1), jnp.int32)
        hi0 = jnp.full((B, 1), 0x3F800001, jnp.int32)
        lo, _ = jax.lax.fori_loop(0, 31, bisect_step, (lo0, hi0))
        return lo

    t = jax.lax.cond(
        jnp.all(cnt_pos < KSEL),
        lambda _: jnp.zeros((B, 1), jnp.int32),
        bisect_all,
        operand=0,
    )

    mask_gt = sb > t
    g = jnp.sum(jnp.where(mask_gt, 1, 0), axis=1, keepdims=True)

    work_ref[:, :] = jnp.where(mask_gt, sc, -1.0)
    max_g = jnp.max(g)
    slot_iota = jax.lax.broadcasted_iota(jnp.int32, (B, KSEL), 1)

    def extract_step(p, gslot):
        w = work_ref[:, :]
        m = jnp.max(w, axis=1, keepdims=True)
        amin = jnp.min(jnp.where(w == m, iota_n, N), axis=1, keepdims=True)
        work_ref[:, :] = jnp.where(iota_n == amin, -1.0, w)
        return jnp.where(slot_iota == p, amin, gslot)

    gslot = jax.lax.fori_loop(
        0, max_g, extract_step, jnp.zeros((B, KSEL), jnp.int32))

    ceq_ref[:, :] = _cumsum_lanes(jnp.where(sb == t, 1, 0))
    lim = slot_iota - g
    CHN = 512

    def nchunk_step(j, acc):
        cc = ceq_ref[:, pl.ds(j * CHN, CHN)]
        part = jnp.sum(
            jnp.where(cc[:, None, :] <= lim[:, :, None], 1, 0), axis=2)
        return acc + part

    pmax = jnp.max(jnp.sum(jnp.where(ceq_ref[:, :] <= KSEL, 1, 0), axis=1))
    nchunks = (pmax + CHN - 1) // CHN
    eqpos = jax.lax.fori_loop(
        0, nchunks, nchunk_step, jnp.zeros((B, KSEL), jnp.int32))

    idx_ref[:, :] = jnp.where(slot_iota < g, gslot, eqpos)


_SC_L = 16
_SC_NV = N // _SC_L


def _sc_topk_body(sc_hbm, out_hbm, row_v, candv_v, candi_v, out_v):
    wid = lax.axis_index("s") * 2 + lax.axis_index("c")

    @pl.when(wid < B)
    def _():
        pltpu.sync_copy(sc_hbm.at[wid], row_v)
        iota16 = lax.broadcasted_iota(jnp.int32, (_SC_L,), 0)

        def vload(i):
            return row_v[pl.ds(i * _SC_L, _SC_L)]

        def cursor_max(vals_ref, nv, lb, li):
            def scan_body(j, acc):
                ab, ai = acc
                v = vals_ref[pl.ds(j * _SC_L, _SC_L)]
                iv = iota16 + j * _SC_L
                elig = (v < lb) | ((v == lb) & (iv > li))
                v = jnp.where(elig, v, -1.0)
                iv = jnp.where(elig, iv, N)
                better = (v > ab) | ((v == ab) & (iv < ai))
                return (jnp.where(better, v, ab), jnp.where(better, iv, ai))

            ab, ai = lax.fori_loop(
                0, nv, scan_body,
                (jnp.full((_SC_L,), -1.0, jnp.float32),
                 jnp.full((_SC_L,), N, jnp.int32)))
            mb = jnp.max(ab)
            mi = jnp.min(jnp.where(ab == mb, ai, N))
            return mb, mi

        def store_slot(p, val):
            plsc.store_scatter(out_v, [jnp.full((_SC_L,), p, jnp.int32)],
                               jnp.full((_SC_L,), val, jnp.int32),
                               mask=iota16 == 0)

        def cnt_body(i, acc):
            return acc + jnp.where(vload(i) > 0.0, 1, 0)

        cnt_pos = jnp.sum(
            lax.fori_loop(0, _SC_NV, cnt_body, jnp.zeros((_SC_L,), jnp.int32)))

        @pl.when(cnt_pos < KSEL)
        def fast_path():
            def comp_body(i, base):
                v = vload(i)
                m = v > 0.0
                pc = plsc.cumsum(jnp.where(m, 1, 0))
                dest = base + pc - 1
                plsc.store_scatter(candv_v, [dest], v, mask=m)
                plsc.store_scatter(candi_v, [dest], iota16 + i * _SC_L,
                                   mask=m)
                return base + jnp.sum(jnp.where(m, 1, 0))

            g = lax.fori_loop(0, _SC_NV, comp_body, 0)

            def pad_body(i, _):
                dest = iota16 + i * _SC_L
                m = dest >= g
                plsc.store_scatter(candv_v, [dest],
                                   jnp.full((_SC_L,), -1.0, jnp.float32),
                                   mask=m)
                return 0

            lax.fori_loop(0, KSEL // _SC_L, pad_body, 0)

            def cext_body(p, carry):
                lb, li = carry

                def scan_body(j, acc):
                    ab, ai = acc
                    v = candv_v[pl.ds(j * _SC_L, _SC_L)]
                    iv = candi_v[pl.ds(j * _SC_L, _SC_L)]
                    elig = (v < lb) | ((v == lb) & (iv > li))
                    v = jnp.where(elig, v, -1.0)
                    iv = jnp.where(elig, iv, N)
                    better = (v > ab) | ((v == ab) & (iv < ai))
                    return (jnp.where(better, v, ab),
                            jnp.where(better, iv, ai))

                ab, ai = lax.fori_loop(
                    0, (g + _SC_L - 1) // _SC_L, scan_body,
                    (jnp.full((_SC_L,), -1.0, jnp.float32),
                     jnp.full((_SC_L,), N, jnp.int32)))
                mb = jnp.max(ab)
                mi = jnp.min(jnp.where(ab == mb, ai, N))
                store_slot(p, mi)
                return (mb, mi)

            lax.fori_loop(0, g, cext_body, (jnp.float32(2.0), -1))

            def eq_cond(carry):
                i, cnt = carry
                return (i < _SC_NV) & (cnt < KSEL - g)

            def eq_body(carry):
                i, cnt = carry
                m = vload(i) == 0.0
                pc = plsc.cumsum(jnp.where(m, 1, 0))
                dest = g + cnt + pc - 1
                plsc.store_scatter(out_v, [dest], iota16 + i * _SC_L,
                                   mask=m & (dest < KSEL))
                return (i + 1, cnt + jnp.sum(jnp.where(m, 1, 0)))

            lax.while_loop(eq_cond, eq_body, (0, 0))

        @pl.when(cnt_pos >= KSEL)
        def general_path():
            def ext_body(p, carry):
                lb, li = carry
                mb, mi = cursor_max(row_v, _SC_NV, lb, li)
                store_slot(p, mi)
                return (mb, mi)

            nslots = jnp.where(cnt_pos >= KSEL, KSEL, 0)
            lax.fori_loop(0, nslots, ext_body, (jnp.float32(2.0), -1))

        pltpu.sync_copy(out_v, out_hbm.at[wid])


def _sc_topk(scores):
    return pl.kernel(
        _sc_topk_body,
        out_type=jax.ShapeDtypeStruct((B, KSEL), jnp.int32),
        mesh=plsc.VectorSubcoreMesh(core_axis_name="c", subcore_axis_name="s"),
        compiler_params=pltpu.CompilerParams(needs_layout_passes=False),
        scratch_types=[
            pltpu.VMEM((N,), jnp.float32),
            pltpu.VMEM((KSEL,), jnp.float32),
            pltpu.VMEM((KSEL,), jnp.int32),
            pltpu.VMEM((KSEL,), jnp.int32),
        ],
    )(scores)


def _compute(x, rt):
    s = pl.pallas_call(
        _matvec_body,
        grid=(N // N_CHUNK,),
        in_specs=[
            pl.BlockSpec((B, N_CHUNK, D), lambda j: (0, j, 0)),
            pl.BlockSpec((1, D), lambda j: (0, 0)),
        ],
        out_specs=pl.BlockSpec((B, N_CHUNK), lambda j: (0, j)),
        out_shape=jax.ShapeDtypeStruct((B, N), jnp.float32),
    )(x, rt[None, :])

    a = pl.pallas_call(
        _iters_body,
        out_shape=jax.ShapeDtypeStruct((B, 1), jnp.float32),
    )(s)

    bfin = -jax.nn.relu(s + a)
    scores = jnp.exp((s + a + bfin) / EPS)

    return _sc_topk(scores)


def kernel(x, routing_token, num_tokens):
    n = x.shape[-2]
    num_tokens = jnp.minimum(num_tokens, n)
    idx = _compute(x, routing_token)
    valid = jnp.arange(KSEL) < num_tokens
    sel_scores = jnp.broadcast_to(
        jnp.where(valid, 1.0, 0.0).astype(jnp.float32), (x.shape[0], KSEL))
    sel_idx = jnp.where(valid, idx, 0)
    return (sel_scores, sel_idx)


# speedup vs baseline: 14.6052x; 14.5236x over previous
import functools

import jax
import jax.numpy as jnp
from jax import lax
from jax.experimental import pallas as pl
from jax.experimental.pallas import tpu as pltpu
from jax.experimental.pallas import tpu_sc as plsc

EPS = 0.1
N_ITERS = 50
K = 8.0

B, N, D = 4, 8192, 768
KSEL = 512
N_CHUNK = 2048


def _matvec_body(x_ref, rt_ref, s_ref):
    x = x_ref[:, :, :].reshape(B * N_CHUNK, D)
    rt = rt_ref[0]
    s = jax.lax.dot_general(
        x, rt[:, None],
        dimension_numbers=(((1,), (0,)), ((), ())),
        preferred_element_type=jnp.float32,
        precision=jax.lax.Precision.DEFAULT,
    )
    s_ref[:, :] = s.reshape(B, N_CHUNK)


def _iters_body(s_ref, a_ref):
    s = s_ref[:, :]
    constant = EPS * jnp.log(K)
    inv_eps = jnp.float32(1.0 / EPS)

    def one_iter(_, a):
        u = jnp.minimum(s + a, 0.0) * inv_eps
        ssum = jnp.sum(jnp.exp(u), axis=-1, keepdims=True)
        return a + (constant - EPS * jnp.log(ssum))

    a = jax.lax.fori_loop(
        0, N_ITERS, one_iter, jnp.zeros((B, 1), jnp.float32))
    a_ref[:, :] = a


def _cumsum_lanes(x):
    n = x.shape[1]
    sh = 1
    while sh < n:
        shifted = jnp.concatenate(
            [jnp.zeros((x.shape[0], sh), x.dtype), x[:, :n - sh]], axis=1)
        x = x + shifted
        sh *= 2
    return x


def _topk_body(sc_ref, idx_ref, work_ref, ceq_ref):
    sc = sc_ref[:, :]
    sb = jax.lax.bitcast_convert_type(sc, jnp.int32)
    iota_n = jax.lax.broadcasted_iota(jnp.int32, (B, N), 1)

    cnt_pos = jnp.sum(jnp.where(sb > 0, 1, 0), axis=1, keepdims=True)

    def bisect_all(_):
        def bisect_step(_, carry):
            lo, hi = carry
            mid = (lo + hi) // 2
            cnt = jnp.sum(jnp.where(sb >= mid, 1, 0), axis=1, keepdims=True)
            ok = cnt >= KSEL
            return (jnp.where(ok, mid, lo), jnp.where(ok, hi, mid))

        lo0 = jnp.zeros((B, 1), jnp.int32)
        hi0 = jnp.full((B, 1), 0x3F800001, jnp.int32)
        lo, _ = jax.lax.fori_loop(0, 31, bisect_step, (lo0, hi0))
        return lo

    t = jax.lax.cond(
        jnp.all(cnt_pos < KSEL),
        lambda _: jnp.zeros((B, 1), jnp.int32),
        bisect_all,
        operand=0,
    )

    mask_gt = sb > t
    g = jnp.sum(jnp.where(mask_gt, 1, 0), axis=1, keepdims=True)

    work_ref[:, :] = jnp.where(mask_gt, sc, -1.0)
    max_g = jnp.max(g)
    slot_iota = jax.lax.broadcasted_iota(jnp.int32, (B, KSEL), 1)

    def extract_step(p, gslot):
        w = work_ref[:, :]
        m = jnp.max(w, axis=1, keepdims=True)
        amin = jnp.min(jnp.where(w == m, iota_n, N), axis=1, keepdims=True)
        work_ref[:, :] = jnp.where(iota_n == amin, -1.0, w)
        return jnp.where(slot_iota == p, amin, gslot)

    gslot = jax.lax.fori_loop(
        0, max_g, extract_step, jnp.zeros((B, KSEL), jnp.int32))

    ceq_ref[:, :] = _cumsum_lanes(jnp.where(sb == t, 1, 0))
    lim = slot_iota - g
    CHN = 512

    def nchunk_step(j, acc):
        cc = ceq_ref[:, pl.ds(j * CHN, CHN)]
        part = jnp.sum(
            jnp.where(cc[:, None, :] <= lim[:, :, None], 1, 0), axis=2)
        return acc + part

    pmax = jnp.max(jnp.sum(jnp.where(ceq_ref[:, :] <= KSEL, 1, 0), axis=1))
    nchunks = (pmax + CHN - 1) // CHN
    eqpos = jax.lax.fori_loop(
        0, nchunks, nchunk_step, jnp.zeros((B, KSEL), jnp.int32))

    idx_ref[:, :] = jnp.where(slot_iota < g, gslot, eqpos)


_SC_L = 16
_SC_NV = N // _SC_L


def _sc_topk_body(sc_hbm, out_hbm, row_v, candv_v, candi_v, out_v):
    wid = lax.axis_index("s") * 2 + lax.axis_index("c")

    @pl.when(wid < B)
    def _():
        pltpu.sync_copy(sc_hbm.at[wid], row_v)
        iota16 = lax.broadcasted_iota(jnp.int32, (_SC_L,), 0)

        def vload(i):
            return row_v[pl.ds(i * _SC_L, _SC_L)]

        def cursor_max(vals_ref, nv, lb, li):
            def scan_body(j, acc):
                ab, ai = acc
                v = vals_ref[pl.ds(j * _SC_L, _SC_L)]
                iv = iota16 + j * _SC_L
                elig = (v < lb) | ((v == lb) & (iv > li))
                v = jnp.where(elig, v, -1.0)
                iv = jnp.where(elig, iv, N)
                better = (v > ab) | ((v == ab) & (iv < ai))
                return (jnp.where(better, v, ab), jnp.where(better, iv, ai))

            ab, ai = lax.fori_loop(
                0, nv, scan_body,
                (jnp.full((_SC_L,), -1.0, jnp.float32),
                 jnp.full((_SC_L,), N, jnp.int32)))
            mb = jnp.max(ab)
            mi = jnp.min(jnp.where(ab == mb, ai, N))
            return mb, mi

        def store_slot(p, val):
            plsc.store_scatter(out_v, [jnp.full((_SC_L,), p, jnp.int32)],
                               jnp.full((_SC_L,), val, jnp.int32),
                               mask=iota16 == 0)

        def cnt_body(i, acc):
            return acc + jnp.where(vload(i) > 0.0, 1, 0)

        cnt_pos = jnp.sum(
            lax.fori_loop(0, _SC_NV, cnt_body, jnp.zeros((_SC_L,), jnp.int32)))

        @pl.when(cnt_pos < KSEL)
        def fast_path():
            def comp_body(i, base):
                v = vload(i)
                m = v > 0.0
                pc = plsc.cumsum(jnp.where(m, 1, 0))
                dest = base + pc - 1
                plsc.store_scatter(candv_v, [dest], v, mask=m)
                plsc.store_scatter(candi_v, [dest], iota16 + i * _SC_L,
                                   mask=m)
                return base + jnp.sum(jnp.where(m, 1, 0))

            g = lax.fori_loop(0, _SC_NV, comp_body, 0)

            def pad_body(i, _):
                dest = iota16 + i * _SC_L
                m = dest >= g
                plsc.store_scatter(candv_v, [dest],
                                   jnp.full((_SC_L,), -1.0, jnp.float32),
                                   mask=m)
                return 0

            lax.fori_loop(0, KSEL // _SC_L, pad_body, 0)

            def cext_body(p, carry):
                lb, li = carry

                def scan_body(j, acc):
                    ab, ai = acc
                    v = candv_v[pl.ds(j * _SC_L, _SC_L)]
                    iv = candi_v[pl.ds(j * _SC_L, _SC_L)]
                    elig = (v < lb) | ((v == lb) & (iv > li))
                    v = jnp.where(elig, v, -1.0)
                    iv = jnp.where(elig, iv, N)
                    better = (v > ab) | ((v == ab) & (iv < ai))
                    return (jnp.where(better, v, ab),
                            jnp.where(better, iv, ai))

                ab, ai = lax.fori_loop(
                    0, (g + _SC_L - 1) // _SC_L, scan_body,
                    (jnp.full((_SC_L,), -1.0, jnp.float32),
                     jnp.full((_SC_L,), N, jnp.int32)))
                mb = jnp.max(ab)
                mi = jnp.min(jnp.where(ab == mb, ai, N))
                store_slot(p, mi)
                return (mb, mi)

            lax.fori_loop(0, g, cext_body, (jnp.float32(2.0), -1))

            def eq_cond(carry):
                i, cnt = carry
                return (i < _SC_NV) & (cnt < KSEL - g)

            def eq_body(carry):
                i, cnt = carry
                m = vload(i) == 0.0
                pc = plsc.cumsum(jnp.where(m, 1, 0))
                dest = g + cnt + pc - 1
                plsc.store_scatter(out_v, [dest], iota16 + i * _SC_L,
                                   mask=m & (dest < KSEL))
                return (i + 1, cnt + jnp.sum(jnp.where(m, 1, 0)))

            lax.while_loop(eq_cond, eq_body, (0, 0))

        @pl.when(cnt_pos >= KSEL)
        def general_path():
            def ext_body(p, carry):
                lb, li = carry
                mb, mi = cursor_max(row_v, _SC_NV, lb, li)
                store_slot(p, mi)
                return (mb, mi)

            nslots = jnp.where(cnt_pos >= KSEL, KSEL, 0)
            lax.fori_loop(0, nslots, ext_body, (jnp.float32(2.0), -1))

        pltpu.sync_copy(out_v, out_hbm.at[wid])


def _sc_topk(scores):
    return pl.kernel(
        _sc_topk_body,
        out_type=jax.ShapeDtypeStruct((B, KSEL), jnp.int32),
        mesh=plsc.VectorSubcoreMesh(core_axis_name="c", subcore_axis_name="s"),
        compiler_params=pltpu.CompilerParams(needs_layout_passes=False),
        scratch_types=[
            pltpu.VMEM((N,), jnp.float32),
            pltpu.VMEM((KSEL,), jnp.float32),
            pltpu.VMEM((KSEL,), jnp.int32),
            pltpu.VMEM((KSEL,), jnp.int32),
        ],
    )(scores)


def _compute(x, rt):
    s = pl.pallas_call(
        _matvec_body,
        grid=(N // N_CHUNK,),
        in_specs=[
            pl.BlockSpec((B, N_CHUNK, D), lambda j: (0, j, 0)),
            pl.BlockSpec((1, D), lambda j: (0, 0)),
        ],
        out_specs=pl.BlockSpec((B, N_CHUNK), lambda j: (0, j)),
        out_shape=jax.ShapeDtypeStruct((B, N), jnp.float32),
    )(x, rt[None, :])

    a = pl.pallas_call(
        _iters_body,
        out_shape=jax.ShapeDtypeStruct((B, 1), jnp.float32),
    )(s)

    bfin = -jax.nn.relu(s + a)
    scores = jnp.exp((s + a + bfin) / EPS)

    return pl.pallas_call(
        _topk_body,
        out_shape=jax.ShapeDtypeStruct((B, KSEL), jnp.int32),
        scratch_shapes=[pltpu.VMEM((B, N), jnp.float32),
                        pltpu.VMEM((B, N), jnp.int32)],
    )(scores)


def kernel(x, routing_token, num_tokens):
    n = x.shape[-2]
    num_tokens = jnp.minimum(num_tokens, n)
    idx = _compute(x, routing_token)
    valid = jnp.arange(KSEL) < num_tokens
    sel_scores = jnp.broadcast_to(
        jnp.where(valid, 1.0, 0.0).astype(jnp.float32), (x.shape[0], KSEL))
    sel_idx = jnp.where(valid, idx, 0)
    return (sel_scores, sel_idx)


# speedup vs baseline: 15.6870x; 1.0741x over previous
import functools

import jax
import jax.numpy as jnp
from jax import lax
from jax.experimental import pallas as pl
from jax.experimental.pallas import tpu as pltpu
from jax.experimental.pallas import tpu_sc as plsc

EPS = 0.1
N_ITERS = 50
K = 8.0

B, N, D = 4, 8192, 768
KSEL = 512
N_CHUNK = 1024


def _matvec_body(x_ref, rt_ref, s_ref):
    x = x_ref[:, :, :].reshape(B * N_CHUNK, D)
    rt = rt_ref[0]
    s = jax.lax.dot_general(
        x, rt[:, None],
        dimension_numbers=(((1,), (0,)), ((), ())),
        preferred_element_type=jnp.float32,
        precision=jax.lax.Precision.DEFAULT,
    )
    s_ref[:, :] = s.reshape(B, N_CHUNK)


def _iters_body(s_ref, out_ref):
    s = s_ref[:, :]
    constant = EPS * jnp.log(K)
    inv_eps = jnp.float32(1.0 / EPS)

    def one_iter(_, a):
        u = jnp.minimum(s + a, 0.0) * inv_eps
        ssum = jnp.sum(jnp.exp(u), axis=-1, keepdims=True)
        return a + (constant - EPS * jnp.log(ssum))

    a = jax.lax.fori_loop(
        0, N_ITERS, one_iter, jnp.zeros((B, 1), jnp.float32))
    t1 = s + a
    bfin = -jax.nn.relu(t1)
    out_ref[:, :] = jnp.exp((t1 + bfin) / EPS)


def _cumsum_lanes(x):
    n = x.shape[1]
    sh = 1
    while sh < n:
        shifted = jnp.concatenate(
            [jnp.zeros((x.shape[0], sh), x.dtype), x[:, :n - sh]], axis=1)
        x = x + shifted
        sh *= 2
    return x


def _topk_body(sc_ref, idx_ref, work_ref, ceq_ref):
    sc = sc_ref[:, :]
    sb = jax.lax.bitcast_convert_type(sc, jnp.int32)
    iota_n = jax.lax.broadcasted_iota(jnp.int32, (B, N), 1)

    cnt_pos = jnp.sum(jnp.where(sb > 0, 1, 0), axis=1, keepdims=True)

    def bisect_all(_):
        def bisect_step(_, carry):
            lo, hi = carry
            mid = (lo + hi) // 2
            cnt = jnp.sum(jnp.where(sb >= mid, 1, 0), axis=1, keepdims=True)
            ok = cnt >= KSEL
            return (jnp.where(ok, mid, lo), jnp.where(ok, hi, mid))

        lo0 = jnp.zeros((B, 1), jnp.int32)
        hi0 = jnp.full((B, 1), 0x3F800001, jnp.int32)
        lo, _ = jax.lax.fori_loop(0, 31, bisect_step, (lo0, hi0))
        return lo

    t = jax.lax.cond(
        jnp.all(cnt_pos < KSEL),
        lambda _: jnp.zeros((B, 1), jnp.int32),
        bisect_all,
        operand=0,
    )

    mask_gt = sb > t
    g = jnp.sum(jnp.where(mask_gt, 1, 0), axis=1, keepdims=True)

    work_ref[:, :] = jnp.where(mask_gt, sc, -1.0)
    max_g = jnp.max(g)
    slot_iota = jax.lax.broadcasted_iota(jnp.int32, (B, KSEL), 1)

    def extract_step(p, gslot):
        w = work_ref[:, :]
        m = jnp.max(w, axis=1, keepdims=True)
        amin = jnp.min(jnp.where(w == m, iota_n, N), axis=1, keepdims=True)
        work_ref[:, :] = jnp.where(iota_n == amin, -1.0, w)
        return jnp.where(slot_iota == p, amin, gslot)

    gslot = jax.lax.fori_loop(
        0, max_g, extract_step, jnp.zeros((B, KSEL), jnp.int32))

    ceq_ref[:, :] = _cumsum_lanes(jnp.where(sb == t, 1, 0))
    lim = slot_iota - g
    CHN = 512

    def nchunk_step(j, acc):
        cc = ceq_ref[:, pl.ds(j * CHN, CHN)]
        part = jnp.sum(
            jnp.where(cc[:, None, :] <= lim[:, :, None], 1, 0), axis=2)
        return acc + part

    pmax = jnp.max(jnp.sum(jnp.where(ceq_ref[:, :] <= KSEL, 1, 0), axis=1))
    nchunks = (pmax + CHN - 1) // CHN
    eqpos = jax.lax.fori_loop(
        0, nchunks, nchunk_step, jnp.zeros((B, KSEL), jnp.int32))

    idx_ref[:, :] = jnp.where(slot_iota < g, gslot, eqpos)


_SC_L = 16
_SC_NV = N // _SC_L


def _sc_topk_body(sc_hbm, out_hbm, row_v, candv_v, candi_v, out_v):
    wid = lax.axis_index("s") * 2 + lax.axis_index("c")

    @pl.when(wid < B)
    def _():
        pltpu.sync_copy(sc_hbm.at[wid], row_v)
        iota16 = lax.broadcasted_iota(jnp.int32, (_SC_L,), 0)

        def vload(i):
            return row_v[pl.ds(i * _SC_L, _SC_L)]

        def cursor_max(vals_ref, nv, lb, li):
            def scan_body(j, acc):
                ab, ai = acc
                v = vals_ref[pl.ds(j * _SC_L, _SC_L)]
                iv = iota16 + j * _SC_L
                elig = (v < lb) | ((v == lb) & (iv > li))
                v = jnp.where(elig, v, -1.0)
                iv = jnp.where(elig, iv, N)
                better = (v > ab) | ((v == ab) & (iv < ai))
                return (jnp.where(better, v, ab), jnp.where(better, iv, ai))

            ab, ai = lax.fori_loop(
                0, nv, scan_body,
                (jnp.full((_SC_L,), -1.0, jnp.float32),
                 jnp.full((_SC_L,), N, jnp.int32)))
            mb = jnp.max(ab)
            mi = jnp.min(jnp.where(ab == mb, ai, N))
            return mb, mi

        def store_slot(p, val):
            plsc.store_scatter(out_v, [jnp.full((_SC_L,), p, jnp.int32)],
                               jnp.full((_SC_L,), val, jnp.int32),
                               mask=iota16 == 0)

        def cnt_body(i, acc):
            return acc + jnp.where(vload(i) > 0.0, 1, 0)

        cnt_pos = jnp.sum(
            lax.fori_loop(0, _SC_NV, cnt_body, jnp.zeros((_SC_L,), jnp.int32)))

        @pl.when(cnt_pos < KSEL)
        def fast_path():
            def comp_body(i, base):
                v = vload(i)
                m = v > 0.0
                pc = plsc.cumsum(jnp.where(m, 1, 0))
                dest = base + pc - 1
                plsc.store_scatter(candv_v, [dest], v, mask=m)
                plsc.store_scatter(candi_v, [dest], iota16 + i * _SC_L,
                                   mask=m)
                return base + jnp.sum(jnp.where(m, 1, 0))

            g = lax.fori_loop(0, _SC_NV, comp_body, 0)

            def pad_body(i, _):
                dest = iota16 + i * _SC_L
                m = dest >= g
                plsc.store_scatter(candv_v, [dest],
                                   jnp.full((_SC_L,), -1.0, jnp.float32),
                                   mask=m)
                return 0

            lax.fori_loop(0, KSEL // _SC_L, pad_body, 0)

            def cext_body(p, carry):
                lb, li = carry

                def scan_body(j, acc):
                    ab, ai = acc
                    v = candv_v[pl.ds(j * _SC_L, _SC_L)]
                    iv = candi_v[pl.ds(j * _SC_L, _SC_L)]
                    elig = (v < lb) | ((v == lb) & (iv > li))
                    v = jnp.where(elig, v, -1.0)
                    iv = jnp.where(elig, iv, N)
                    better = (v > ab) | ((v == ab) & (iv < ai))
                    return (jnp.where(better, v, ab),
                            jnp.where(better, iv, ai))

                ab, ai = lax.fori_loop(
                    0, (g + _SC_L - 1) // _SC_L, scan_body,
                    (jnp.full((_SC_L,), -1.0, jnp.float32),
                     jnp.full((_SC_L,), N, jnp.int32)))
                mb = jnp.max(ab)
                mi = jnp.min(jnp.where(ab == mb, ai, N))
                store_slot(p, mi)
                return (mb, mi)

            lax.fori_loop(0, g, cext_body, (jnp.float32(2.0), -1))

            def eq_cond(carry):
                i, cnt = carry
                return (i < _SC_NV) & (cnt < KSEL - g)

            def eq_body(carry):
                i, cnt = carry
                m = vload(i) == 0.0
                pc = plsc.cumsum(jnp.where(m, 1, 0))
                dest = g + cnt + pc - 1
                plsc.store_scatter(out_v, [dest], iota16 + i * _SC_L,
                                   mask=m & (dest < KSEL))
                return (i + 1, cnt + jnp.sum(jnp.where(m, 1, 0)))

            lax.while_loop(eq_cond, eq_body, (0, 0))

        @pl.when(cnt_pos >= KSEL)
        def general_path():
            def ext_body(p, carry):
                lb, li = carry
                mb, mi = cursor_max(row_v, _SC_NV, lb, li)
                store_slot(p, mi)
                return (mb, mi)

            nslots = jnp.where(cnt_pos >= KSEL, KSEL, 0)
            lax.fori_loop(0, nslots, ext_body, (jnp.float32(2.0), -1))

        pltpu.sync_copy(out_v, out_hbm.at[wid])


def _sc_topk(scores):
    return pl.kernel(
        _sc_topk_body,
        out_type=jax.ShapeDtypeStruct((B, KSEL), jnp.int32),
        mesh=plsc.VectorSubcoreMesh(core_axis_name="c", subcore_axis_name="s"),
        compiler_params=pltpu.CompilerParams(needs_layout_passes=False),
        scratch_types=[
            pltpu.VMEM((N,), jnp.float32),
            pltpu.VMEM((KSEL,), jnp.float32),
            pltpu.VMEM((KSEL,), jnp.int32),
            pltpu.VMEM((KSEL,), jnp.int32),
        ],
    )(scores)


def _compute(x, rt):
    s = pl.pallas_call(
        _matvec_body,
        grid=(N // N_CHUNK,),
        in_specs=[
            pl.BlockSpec((B, N_CHUNK, D), lambda j: (0, j, 0)),
            pl.BlockSpec((1, D), lambda j: (0, 0)),
        ],
        out_specs=pl.BlockSpec((B, N_CHUNK), lambda j: (0, j)),
        out_shape=jax.ShapeDtypeStruct((B, N), jnp.float32),
    )(x, rt[None, :])

    scores = pl.pallas_call(
        _iters_body,
        out_shape=jax.ShapeDtypeStruct((B, N), jnp.float32),
    )(s)

    return pl.pallas_call(
        _topk_body,
        out_shape=jax.ShapeDtypeStruct((B, KSEL), jnp.int32),
        scratch_shapes=[pltpu.VMEM((B, N), jnp.float32),
                        pltpu.VMEM((B, N), jnp.int32)],
    )(scores)


def kernel(x, routing_token, num_tokens):
    n = x.shape[-2]
    num_tokens = jnp.minimum(num_tokens, n)
    idx = _compute(x, routing_token)
    valid = jnp.arange(KSEL) < num_tokens
    sel_scores = jnp.broadcast_to(
        jnp.where(valid, 1.0, 0.0).astype(jnp.float32), (x.shape[0], KSEL))
    sel_idx = jnp.where(valid, idx, 0)
    return (sel_scores, sel_idx)


# speedup vs baseline: 16.3691x; 1.0435x over previous
import jax
import jax.numpy as jnp
from jax import lax
from jax.experimental import pallas as pl
from jax.experimental.pallas import tpu as pltpu

EPS = 0.1
N_ITERS = 50
K = 8.0

B, N, D = 4, 8192, 768
KSEL = 512
N_CHUNK = 1024
N_STEPS = N // N_CHUNK


def _scores_from_s(s):
    constant = EPS * jnp.log(K)
    inv_eps = jnp.float32(1.0 / EPS)

    def one_iter(_, a):
        u = jnp.minimum(s + a, 0.0) * inv_eps
        ssum = jnp.sum(jnp.exp(u), axis=-1, keepdims=True)
        return a + (constant - EPS * jnp.log(ssum))

    a = lax.fori_loop(0, N_ITERS, one_iter, jnp.zeros((B, 1), jnp.float32))
    t1 = s + a
    bfin = -jax.nn.relu(t1)
    return jnp.exp((t1 + bfin) / EPS)


def _cumsum_lanes(x):
    n = x.shape[1]
    sh = 1
    while sh < n:
        shifted = jnp.concatenate(
            [jnp.zeros((x.shape[0], sh), x.dtype), x[:, :n - sh]], axis=1)
        x = x + shifted
        sh *= 2
    return x


def _topk_indices(sc, idx_ref, work_ref, ceq_ref):
    sb = lax.bitcast_convert_type(sc, jnp.int32)
    iota_n = lax.broadcasted_iota(jnp.int32, (B, N), 1)

    cnt_pos = jnp.sum(jnp.where(sb > 0, 1, 0), axis=1, keepdims=True)

    def bisect_all(_):
        def bisect_step(_, carry):
            lo, hi = carry
            mid = (lo + hi) // 2
            cnt = jnp.sum(jnp.where(sb >= mid, 1, 0), axis=1, keepdims=True)
            ok = cnt >= KSEL
            return (jnp.where(ok, mid, lo), jnp.where(ok, hi, mid))

        lo0 = jnp.zeros((B, 1), jnp.int32)
        hi0 = jnp.full((B, 1), 0x3F800001, jnp.int32)
        lo, _ = lax.fori_loop(0, 31, bisect_step, (lo0, hi0))
        return lo

    t = lax.cond(
        jnp.all(cnt_pos < KSEL),
        lambda _: jnp.zeros((B, 1), jnp.int32),
        bisect_all,
        operand=0,
    )

    mask_gt = sb > t
    g = jnp.sum(jnp.where(mask_gt, 1, 0), axis=1, keepdims=True)

    work_ref[:, :] = jnp.where(mask_gt, sc, -1.0)
    max_g = jnp.max(g)
    slot_iota = lax.broadcasted_iota(jnp.int32, (B, KSEL), 1)

    def extract_step(p, gslot):
        w = work_ref[:, :]
        m = jnp.max(w, axis=1, keepdims=True)
        amin = jnp.min(jnp.where(w == m, iota_n, N), axis=1, keepdims=True)
        work_ref[:, :] = jnp.where(iota_n == amin, -1.0, w)
        return jnp.where(slot_iota == p, amin, gslot)

    gslot = lax.fori_loop(
        0, max_g, extract_step, jnp.zeros((B, KSEL), jnp.int32))

    ceq_ref[:, :] = _cumsum_lanes(jnp.where(sb == t, 1, 0))
    lim = slot_iota - g
    CHN = 512

    def nchunk_step(j, acc):
        cc = ceq_ref[:, pl.ds(j * CHN, CHN)]
        part = jnp.sum(
            jnp.where(cc[:, None, :] <= lim[:, :, None], 1, 0), axis=2)
        return acc + part

    pmax = jnp.max(jnp.sum(jnp.where(ceq_ref[:, :] <= KSEL, 1, 0), axis=1))
    nchunks = (pmax + CHN - 1) // CHN
    eqpos = lax.fori_loop(
        0, nchunks, nchunk_step, jnp.zeros((B, KSEL), jnp.int32))

    idx_ref[:, :] = jnp.where(slot_iota < g, gslot, eqpos)


def _fused_body(x_ref, rt_ref, idx_ref, s_scr, work_scr, ceq_scr):
    j = pl.program_id(0)
    x = x_ref[:, :, :].reshape(B * N_CHUNK, D)
    rt = rt_ref[0]
    sv = lax.dot_general(
        x, rt[:, None],
        dimension_numbers=(((1,), (0,)), ((), ())),
        preferred_element_type=jnp.float32,
        precision=lax.Precision.DEFAULT,
    )
    s_scr[:, pl.ds(j * N_CHUNK, N_CHUNK)] = sv.reshape(B, N_CHUNK)

    @pl.when(j == N_STEPS - 1)
    def _():
        scores = _scores_from_s(s_scr[:, :])
        _topk_indices(scores, idx_ref, work_scr, ceq_scr)


def _compute(x, rt):
    return pl.pallas_call(
        _fused_body,
        grid=(N_STEPS,),
        in_specs=[
            pl.BlockSpec((B, N_CHUNK, D), lambda j: (0, j, 0)),
            pl.BlockSpec((1, D), lambda j: (0, 0)),
        ],
        out_specs=pl.BlockSpec((B, KSEL), lambda j: (0, 0)),
        out_shape=jax.ShapeDtypeStruct((B, KSEL), jnp.int32),
        scratch_shapes=[
            pltpu.VMEM((B, N), jnp.float32),
            pltpu.VMEM((B, N), jnp.float32),
            pltpu.VMEM((B, N), jnp.int32),
        ],
    )(x, rt[None, :])


def kernel(x, routing_token, num_tokens):
    n = x.shape[-2]
    num_tokens = jnp.minimum(num_tokens, n)
    idx = _compute(x, routing_token)
    valid = jnp.arange(KSEL) < num_tokens
    sel_scores = jnp.broadcast_to(
        jnp.where(valid, 1.0, 0.0).astype(jnp.float32), (x.shape[0], KSEL))
    sel_idx = jnp.where(valid, idx, 0)
    return (sel_scores, sel_idx)
